# Initial kernel scaffold; baseline (speedup 1.0000x reference)
import jax
import jax.numpy as jnp
from jax.experimental import pallas as pl


def kernel(pred_noise, target_noise, generated_points, original_points, content_original, content_from_noisy, style_source, style_target):
    raise NotImplementedError("write your pallas kernel here")


# speedup vs baseline: 23.1708x; 23.1708x over previous
import jax
import jax.numpy as jnp
from jax.experimental import pallas as pl
from jax.experimental.pallas import tpu as pltpu

B = 8
NPTS = 2048
CROWS = 256
CDIM = 2048
SDIM = 256
RBLK = 128
NBLK = NPTS // RBLK
CCH = CROWS // NBLK
BIG = 3.0e38


def _loss_body(pred_ref, targ_ref, g_ref, o_ref, gt_ref, ot_ref, co_ref, cn_ref,
               ss_ref, st_ref,
               diff2_ref, cmse_ref, varo_ref, varn_ref, abso_ref, style_ref,
               local_ref, smd_ref, smd2_ref, cen_ref, rng_ref, covf_ref,
               sgd_ref, sgd2_ref, sod_ref, sod2_ref):
    b = pl.program_id(0)
    rb = pl.program_id(1)

    @pl.when(jnp.logical_and(b == 0, rb == 0))
    def _init_global():
        for r in (diff2_ref, cmse_ref, varo_ref, varn_ref, abso_ref,
                  style_ref, local_ref):
            r[0, 0] = 0.0

    @pl.when(rb == 0)
    def _init_batch():
        for r in (smd_ref, smd2_ref, cen_ref, rng_ref, covf_ref,
                  sgd_ref, sgd2_ref, sod_ref, sod2_ref):
            r[0, 0, 0] = 0.0

    dd = pred_ref[0] - targ_ref[0]
    diff2_ref[0, 0] += jnp.sum(dd * dd)

    co = co_ref[0]
    cn = cn_ref[0]
    dc = cn - co
    cmse_ref[0, 0] += jnp.sum(dc * dc)
    so = jnp.sum(co, axis=1)
    so2 = jnp.sum(co * co, axis=1)
    varo_ref[0, 0] += jnp.sum((so2 - so * so * (1.0 / CDIM)) * (1.0 / (CDIM - 1)))
    sn = jnp.sum(cn, axis=1)
    sn2 = jnp.sum(cn * cn, axis=1)
    varn_ref[0, 0] += jnp.sum((sn2 - sn * sn * (1.0 / CDIM)) * (1.0 / (CDIM - 1)))
    abso_ref[0, 0] += jnp.sum(jnp.abs(co))

    @pl.when(rb == 0)
    def _style():
        s = ss_ref[0]
        t = st_ref[0]
        num = jnp.sum(s * t)
        ns = jnp.maximum(jnp.sqrt(jnp.sum(s * s)), 1e-8)
        nt = jnp.maximum(jnp.sqrt(jnp.sum(t * t)), 1e-8)
        style_ref[0, 0] += jnp.abs(num / (ns * nt) - 0.5)

    gt = gt_ref[0]
    ot = ot_ref[0]
    g_full = g_ref[0]
    o_full = o_ref[0]

    @pl.when(rb == 0)
    def _moments():
        gs31 = jnp.sum(gt, axis=1, keepdims=True)
        os31 = jnp.sum(ot, axis=1, keepdims=True)
        gs13 = jnp.sum(g_full, axis=0, keepdims=True)
        os13 = jnp.sum(o_full, axis=0, keepdims=True)
        inv_n = 1.0 / NPTS
        gc31, oc31 = gs31 * inv_n, os31 * inv_n
        gc13, oc13 = gs13 * inv_n, os13 * inv_n
        dcn = gc31 - oc31
        cen_ref[0, 0, 0] = jnp.sum(dcn * dcn)
        rg = (jnp.max(gt, axis=1, keepdims=True) - jnp.min(gt, axis=1, keepdims=True)) \
           - (jnp.max(ot, axis=1, keepdims=True) - jnp.min(ot, axis=1, keepdims=True))
        rng_ref[0, 0, 0] = jnp.sum(rg * rg)
        mg = jax.lax.dot_general(gt, g_full, (((1,), (0,)), ((), ())),
                                 preferred_element_type=jnp.float32)
        mo = jax.lax.dot_general(ot, o_full, (((1,), (0,)), ((), ())),
                                 preferred_element_type=jnp.float32)
        inv_nm1 = 1.0 / (NPTS - 1)
        gcov = (mg - (gs31 * gc13)) * inv_nm1
        ocov = (mo - (os31 * oc13)) * inv_nm1
        dcv = gcov - ocov
        covf_ref[0, 0, 0] = jnp.sqrt(jnp.sum(dcv * dcv))
        gcen = gt - gc31
        ocen = ot - oc31
        gn2 = jnp.sum(gcen * gcen, axis=0, keepdims=True)
        on2 = jnp.sum(ocen * ocen, axis=0, keepdims=True)
        sgd_ref[0, 0, 0] = jnp.sum(jnp.sqrt(gn2))
        sgd2_ref[0, 0, 0] = jnp.sum(gn2)
        sod_ref[0, 0, 0] = jnp.sum(jnp.sqrt(on2))
        sod2_ref[0, 0, 0] = jnp.sum(on2)

    row0 = rb * RBLK
    o_blk = o_ref[0, pl.ds(row0, RBLK), :]
    g_blk = g_ref[0, pl.ds(row0, RBLK), :]
    h_blk = g_blk - o_blk
    ht = gt - ot

    def _cdist2(blk, full_t):
        acc = (blk[:, 0:1] - full_t[0:1, :]) ** 2
        acc += (blk[:, 1:2] - full_t[1:2, :]) ** 2
        acc += (blk[:, 2:3] - full_t[2:3, :]) ** 2
        return acc

    d2o = _cdist2(o_blk, ot)
    pmat = _cdist2(h_blk, ht)
    d2g = _cdist2(g_blk, gt)

    def _kth_threshold(mat, k):
        sel = mat
        for _ in range(k - 1):
            m = jnp.min(sel, axis=1, keepdims=True)
            sel = jnp.where(sel <= m, BIG, sel)
        return jnp.min(sel, axis=1, keepdims=True)

    t17 = _kth_threshold(d2o, 17)
    local_ref[0, 0] += jnp.sum(jnp.where(d2o <= t17, pmat, 0.0))

    t5 = _kth_threshold(d2g, 5)
    srow = jnp.sum(jnp.where(d2g <= t5, jnp.sqrt(d2g), 0.0), axis=1,
                   keepdims=True)
    md = srow * 0.25
    smd_ref[0, 0, 0] += jnp.sum(md)
    smd2_ref[0, 0, 0] += jnp.sum(md * md)


def kernel(pred_noise, target_noise, generated_points, original_points,
           content_original, content_from_noisy, style_source, style_target):
    gt = jnp.transpose(generated_points, (0, 2, 1))
    ot = jnp.transpose(original_points, (0, 2, 1))
    ss = style_source[:, None, :]
    st = style_target[:, None, :]

    smem11 = pl.BlockSpec((1, 1), lambda b, rb: (0, 0), memory_space=pltpu.SMEM)
    smemb1 = pl.BlockSpec((1, 1, 1), lambda b, rb: (b, 0, 0),
                          memory_space=pltpu.SMEM)
    f32 = jnp.float32

    out_shape = ([jax.ShapeDtypeStruct((1, 1), f32)] * 7
                 + [jax.ShapeDtypeStruct((B, 1, 1), f32)] * 9)
    out_specs = [smem11] * 7 + [smemb1] * 9

    in_specs = [
        pl.BlockSpec((1, RBLK, 3), lambda b, rb: (b, rb, 0)),
        pl.BlockSpec((1, RBLK, 3), lambda b, rb: (b, rb, 0)),
        pl.BlockSpec((1, NPTS, 3), lambda b, rb: (b, 0, 0)),
        pl.BlockSpec((1, NPTS, 3), lambda b, rb: (b, 0, 0)),
        pl.BlockSpec((1, 3, NPTS), lambda b, rb: (b, 0, 0)),
        pl.BlockSpec((1, 3, NPTS), lambda b, rb: (b, 0, 0)),
        pl.BlockSpec((1, CCH, CDIM), lambda b, rb: (b, rb, 0)),
        pl.BlockSpec((1, CCH, CDIM), lambda b, rb: (b, rb, 0)),
        pl.BlockSpec((1, 1, SDIM), lambda b, rb: (b, 0, 0)),
        pl.BlockSpec((1, 1, SDIM), lambda b, rb: (b, 0, 0)),
    ]

    outs = pl.pallas_call(
        _loss_body,
        grid=(B, NBLK),
        in_specs=in_specs,
        out_specs=out_specs,
        out_shape=out_shape,
    )(pred_noise, target_noise, generated_points, original_points,
      gt, ot, content_original, content_from_noisy, ss, st)

    (diff2, cmse, varo, varn, abso, style, local,
     smd, smd2, cen, rng, covf, sgd, sgd2, sod, sod2) = outs

    diff_loss = diff2[0, 0] / (B * NPTS * 3)

    c_mse = cmse[0, 0] / (B * CROWS * CDIM)
    orig_var = varo[0, 0] / (B * CROWS)
    noisy_var = varn[0, 0] / (B * CROWS)
    var_loss = jax.nn.relu(0.1 - orig_var) + jax.nn.relu(0.1 - noisy_var)
    act_loss = jax.nn.relu(1.0 - abso[0, 0] / (B * CROWS * CDIM)) * 0.1
    content_loss = c_mse + var_loss + act_loss

    center_loss = jnp.sum(cen) / (B * 3)
    range_loss = jnp.sum(rng) / (B * 3)
    cov_loss = jnp.mean(covf)
    mgd = sgd[:, 0, 0] / NPTS
    mod = sod[:, 0, 0] / NPTS
    stdg = jnp.sqrt((sgd2[:, 0, 0] - sgd[:, 0, 0] ** 2 / NPTS) / (NPTS - 1))
    stdo = jnp.sqrt((sod2[:, 0, 0] - sod[:, 0, 0] ** 2 / NPTS) / (NPTS - 1))
    dist_loss = jnp.mean((mgd - mod) ** 2) + jnp.mean((stdg - stdo) ** 2)
    shape_loss = center_loss + range_loss * 0.5 + cov_loss * 0.1 + dist_loss * 0.5

    local_loss = local[0, 0] / (B * NPTS * 16 * 3)

    mmd = smd[:, 0, 0] / NPTS
    stds = jnp.sqrt((smd2[:, 0, 0] - smd[:, 0, 0] ** 2 / NPTS) / (NPTS - 1))
    smooth_loss = jnp.mean(stds)
    del mmd

    style_loss = style[0, 0] / B

    total = (diff_loss + content_loss + 2.0 * shape_loss
             + local_loss + 0.5 * smooth_loss + 0.1 * style_loss)
    return total


# speedup vs baseline: 27.1714x; 1.1727x over previous
import jax
import jax.numpy as jnp
from jax.experimental import pallas as pl
from jax.experimental.pallas import tpu as pltpu

B = 8
NPTS = 2048
CROWS = 256
CDIM = 2048
SDIM = 256
RBLK = 512
NBLK = NPTS // RBLK
CCH = CROWS // NBLK
BIG = 3.0e38


def _loss_body(pred_ref, targ_ref, g_ref, o_ref, gt_ref, ot_ref, co_ref, cn_ref,
               ss_ref, st_ref,
               diff2_ref, cmse_ref, varo_ref, varn_ref, abso_ref, style_ref,
               local_ref, smd_ref, smd2_ref, cen_ref, rng_ref, covf_ref,
               sgd_ref, sgd2_ref, sod_ref, sod2_ref):
    b = pl.program_id(0)
    rb = pl.program_id(1)

    @pl.when(jnp.logical_and(b == 0, rb == 0))
    def _init_global():
        for r in (diff2_ref, cmse_ref, varo_ref, varn_ref, abso_ref,
                  style_ref, local_ref):
            r[0, 0] = 0.0

    @pl.when(rb == 0)
    def _init_batch():
        for r in (smd_ref, smd2_ref, cen_ref, rng_ref, covf_ref,
                  sgd_ref, sgd2_ref, sod_ref, sod2_ref):
            r[0, 0, 0] = 0.0

    dd = pred_ref[0] - targ_ref[0]
    diff2_ref[0, 0] += jnp.sum(dd * dd)

    co = co_ref[0]
    cn = cn_ref[0]
    dc = cn - co
    cmse_ref[0, 0] += jnp.sum(dc * dc)
    so = jnp.sum(co, axis=1)
    so2 = jnp.sum(co * co, axis=1)
    varo_ref[0, 0] += jnp.sum((so2 - so * so * (1.0 / CDIM)) * (1.0 / (CDIM - 1)))
    sn = jnp.sum(cn, axis=1)
    sn2 = jnp.sum(cn * cn, axis=1)
    varn_ref[0, 0] += jnp.sum((sn2 - sn * sn * (1.0 / CDIM)) * (1.0 / (CDIM - 1)))
    abso_ref[0, 0] += jnp.sum(jnp.abs(co))

    @pl.when(rb == 0)
    def _style():
        s = ss_ref[0]
        t = st_ref[0]
        num = jnp.sum(s * t)
        ns = jnp.maximum(jnp.sqrt(jnp.sum(s * s)), 1e-8)
        nt = jnp.maximum(jnp.sqrt(jnp.sum(t * t)), 1e-8)
        style_ref[0, 0] += jnp.abs(num / (ns * nt) - 0.5)

    gt = gt_ref[0]
    ot = ot_ref[0]
    g_full = g_ref[0]
    o_full = o_ref[0]

    @pl.when(rb == 0)
    def _moments():
        gs31 = jnp.sum(gt, axis=1, keepdims=True)
        os31 = jnp.sum(ot, axis=1, keepdims=True)
        gs13 = jnp.sum(g_full, axis=0, keepdims=True)
        os13 = jnp.sum(o_full, axis=0, keepdims=True)
        inv_n = 1.0 / NPTS
        gc31, oc31 = gs31 * inv_n, os31 * inv_n
        gc13, oc13 = gs13 * inv_n, os13 * inv_n
        dcn = gc31 - oc31
        cen_ref[0, 0, 0] = jnp.sum(dcn * dcn)
        rg = (jnp.max(gt, axis=1, keepdims=True) - jnp.min(gt, axis=1, keepdims=True)) \
           - (jnp.max(ot, axis=1, keepdims=True) - jnp.min(ot, axis=1, keepdims=True))
        rng_ref[0, 0, 0] = jnp.sum(rg * rg)
        mg = jax.lax.dot_general(gt, g_full, (((1,), (0,)), ((), ())),
                                 preferred_element_type=jnp.float32)
        mo = jax.lax.dot_general(ot, o_full, (((1,), (0,)), ((), ())),
                                 preferred_element_type=jnp.float32)
        inv_nm1 = 1.0 / (NPTS - 1)
        gcov = (mg - (gs31 * gc13)) * inv_nm1
        ocov = (mo - (os31 * oc13)) * inv_nm1
        dcv = gcov - ocov
        covf_ref[0, 0, 0] = jnp.sqrt(jnp.sum(dcv * dcv))
        gcen = gt - gc31
        ocen = ot - oc31
        gn2 = jnp.sum(gcen * gcen, axis=0, keepdims=True)
        on2 = jnp.sum(ocen * ocen, axis=0, keepdims=True)
        sgd_ref[0, 0, 0] = jnp.sum(jnp.sqrt(gn2))
        sgd2_ref[0, 0, 0] = jnp.sum(gn2)
        sod_ref[0, 0, 0] = jnp.sum(jnp.sqrt(on2))
        sod2_ref[0, 0, 0] = jnp.sum(on2)

    row0 = rb * RBLK
    o_blk = o_ref[0, pl.ds(row0, RBLK), :]
    g_blk = g_ref[0, pl.ds(row0, RBLK), :]
    h_blk = g_blk - o_blk
    ht = gt - ot

    def _cdist2(blk, full_t):
        acc = (blk[:, 0:1] - full_t[0:1, :]) ** 2
        acc += (blk[:, 1:2] - full_t[1:2, :]) ** 2
        acc += (blk[:, 2:3] - full_t[2:3, :]) ** 2
        return acc

    d2o = _cdist2(o_blk, ot)
    pmat = _cdist2(h_blk, ht)
    d2g = _cdist2(g_blk, gt)

    def _kth_threshold(mat, k):
        sel = mat
        for _ in range(k - 1):
            m = jnp.min(sel, axis=1, keepdims=True)
            sel = jnp.where(sel <= m, BIG, sel)
        return jnp.min(sel, axis=1, keepdims=True)

    t17 = _kth_threshold(d2o, 17)
    local_ref[0, 0] += jnp.sum(jnp.where(d2o <= t17, pmat, 0.0))

    t5 = _kth_threshold(d2g, 5)
    srow = jnp.sum(jnp.where(d2g <= t5, jnp.sqrt(d2g), 0.0), axis=1,
                   keepdims=True)
    md = srow * 0.25
    smd_ref[0, 0, 0] += jnp.sum(md)
    smd2_ref[0, 0, 0] += jnp.sum(md * md)


def kernel(pred_noise, target_noise, generated_points, original_points,
           content_original, content_from_noisy, style_source, style_target):
    gt = jnp.transpose(generated_points, (0, 2, 1))
    ot = jnp.transpose(original_points, (0, 2, 1))
    ss = style_source[:, None, :]
    st = style_target[:, None, :]

    smem11 = pl.BlockSpec((1, 1), lambda b, rb: (0, 0), memory_space=pltpu.SMEM)
    smemb1 = pl.BlockSpec((1, 1, 1), lambda b, rb: (b, 0, 0),
                          memory_space=pltpu.SMEM)
    f32 = jnp.float32

    out_shape = ([jax.ShapeDtypeStruct((1, 1), f32)] * 7
                 + [jax.ShapeDtypeStruct((B, 1, 1), f32)] * 9)
    out_specs = [smem11] * 7 + [smemb1] * 9

    in_specs = [
        pl.BlockSpec((1, RBLK, 3), lambda b, rb: (b, rb, 0)),
        pl.BlockSpec((1, RBLK, 3), lambda b, rb: (b, rb, 0)),
        pl.BlockSpec((1, NPTS, 3), lambda b, rb: (b, 0, 0)),
        pl.BlockSpec((1, NPTS, 3), lambda b, rb: (b, 0, 0)),
        pl.BlockSpec((1, 3, NPTS), lambda b, rb: (b, 0, 0)),
        pl.BlockSpec((1, 3, NPTS), lambda b, rb: (b, 0, 0)),
        pl.BlockSpec((1, CCH, CDIM), lambda b, rb: (b, rb, 0)),
        pl.BlockSpec((1, CCH, CDIM), lambda b, rb: (b, rb, 0)),
        pl.BlockSpec((1, 1, SDIM), lambda b, rb: (b, 0, 0)),
        pl.BlockSpec((1, 1, SDIM), lambda b, rb: (b, 0, 0)),
    ]

    outs = pl.pallas_call(
        _loss_body,
        grid=(B, NBLK),
        in_specs=in_specs,
        out_specs=out_specs,
        out_shape=out_shape,
    )(pred_noise, target_noise, generated_points, original_points,
      gt, ot, content_original, content_from_noisy, ss, st)

    (diff2, cmse, varo, varn, abso, style, local,
     smd, smd2, cen, rng, covf, sgd, sgd2, sod, sod2) = outs

    diff_loss = diff2[0, 0] / (B * NPTS * 3)

    c_mse = cmse[0, 0] / (B * CROWS * CDIM)
    orig_var = varo[0, 0] / (B * CROWS)
    noisy_var = varn[0, 0] / (B * CROWS)
    var_loss = jax.nn.relu(0.1 - orig_var) + jax.nn.relu(0.1 - noisy_var)
    act_loss = jax.nn.relu(1.0 - abso[0, 0] / (B * CROWS * CDIM)) * 0.1
    content_loss = c_mse + var_loss + act_loss

    center_loss = jnp.sum(cen) / (B * 3)
    range_loss = jnp.sum(rng) / (B * 3)
    cov_loss = jnp.mean(covf)
    mgd = sgd[:, 0, 0] / NPTS
    mod = sod[:, 0, 0] / NPTS
    stdg = jnp.sqrt((sgd2[:, 0, 0] - sgd[:, 0, 0] ** 2 / NPTS) / (NPTS - 1))
    stdo = jnp.sqrt((sod2[:, 0, 0] - sod[:, 0, 0] ** 2 / NPTS) / (NPTS - 1))
    dist_loss = jnp.mean((mgd - mod) ** 2) + jnp.mean((stdg - stdo) ** 2)
    shape_loss = center_loss + range_loss * 0.5 + cov_loss * 0.1 + dist_loss * 0.5

    local_loss = local[0, 0] / (B * NPTS * 16 * 3)

    mmd = smd[:, 0, 0] / NPTS
    stds = jnp.sqrt((smd2[:, 0, 0] - smd[:, 0, 0] ** 2 / NPTS) / (NPTS - 1))
    smooth_loss = jnp.mean(stds)
    del mmd

    style_loss = style[0, 0] / B

    total = (diff_loss + content_loss + 2.0 * shape_loss
             + local_loss + 0.5 * smooth_loss + 0.1 * style_loss)
    return total


# speedup vs baseline: 30.8213x; 1.1343x over previous
import jax
import jax.numpy as jnp
from jax.experimental import pallas as pl
from jax.experimental.pallas import tpu as pltpu

B = 8
NPTS = 2048
CROWS = 256
CDIM = 2048
SDIM = 256
RBLK = 512
NBLK = NPTS // RBLK
CCH = CROWS // NBLK
BIG = 3.0e38


def _loss_body(pred_ref, targ_ref, g_ref, o_ref, gt_ref, ot_ref, co_ref, cn_ref,
               ss_ref, st_ref,
               diff2_ref, cmse_ref, varo_ref, varn_ref, abso_ref, style_ref,
               local_ref, smd_ref, smd2_ref, cen_ref, rng_ref, covf_ref,
               sgd_ref, sgd2_ref, sod_ref, sod2_ref):
    b = pl.program_id(0)
    rb = pl.program_id(1)

    @pl.when(jnp.logical_and(b == 0, rb == 0))
    def _init_global():
        for r in (diff2_ref, cmse_ref, varo_ref, varn_ref, abso_ref,
                  style_ref, local_ref):
            r[0, 0] = 0.0

    @pl.when(rb == 0)
    def _init_batch():
        for r in (smd_ref, smd2_ref, cen_ref, rng_ref, covf_ref,
                  sgd_ref, sgd2_ref, sod_ref, sod2_ref):
            r[0, 0, 0] = 0.0

    dd = pred_ref[0] - targ_ref[0]
    diff2_ref[0, 0] += jnp.sum(dd * dd)

    co = co_ref[0]
    cn = cn_ref[0]
    dc = cn - co
    cmse_ref[0, 0] += jnp.sum(dc * dc)
    so = jnp.sum(co, axis=1)
    so2 = jnp.sum(co * co, axis=1)
    varo_ref[0, 0] += jnp.sum((so2 - so * so * (1.0 / CDIM)) * (1.0 / (CDIM - 1)))
    sn = jnp.sum(cn, axis=1)
    sn2 = jnp.sum(cn * cn, axis=1)
    varn_ref[0, 0] += jnp.sum((sn2 - sn * sn * (1.0 / CDIM)) * (1.0 / (CDIM - 1)))
    abso_ref[0, 0] += jnp.sum(jnp.abs(co))

    @pl.when(rb == 0)
    def _style():
        s = ss_ref[0]
        t = st_ref[0]
        num = jnp.sum(s * t)
        ns = jnp.maximum(jnp.sqrt(jnp.sum(s * s)), 1e-8)
        nt = jnp.maximum(jnp.sqrt(jnp.sum(t * t)), 1e-8)
        style_ref[0, 0] += jnp.abs(num / (ns * nt) - 0.5)

    gt = gt_ref[0]
    ot = ot_ref[0]
    g_full = g_ref[0]
    o_full = o_ref[0]

    @pl.when(rb == 0)
    def _moments():
        gs31 = jnp.sum(gt, axis=1, keepdims=True)
        os31 = jnp.sum(ot, axis=1, keepdims=True)
        gs13 = jnp.sum(g_full, axis=0, keepdims=True)
        os13 = jnp.sum(o_full, axis=0, keepdims=True)
        inv_n = 1.0 / NPTS
        gc31, oc31 = gs31 * inv_n, os31 * inv_n
        gc13, oc13 = gs13 * inv_n, os13 * inv_n
        dcn = gc31 - oc31
        cen_ref[0, 0, 0] = jnp.sum(dcn * dcn)
        rg = (jnp.max(gt, axis=1, keepdims=True) - jnp.min(gt, axis=1, keepdims=True)) \
           - (jnp.max(ot, axis=1, keepdims=True) - jnp.min(ot, axis=1, keepdims=True))
        rng_ref[0, 0, 0] = jnp.sum(rg * rg)
        mg = jax.lax.dot_general(gt, g_full, (((1,), (0,)), ((), ())),
                                 preferred_element_type=jnp.float32)
        mo = jax.lax.dot_general(ot, o_full, (((1,), (0,)), ((), ())),
                                 preferred_element_type=jnp.float32)
        inv_nm1 = 1.0 / (NPTS - 1)
        gcov = (mg - (gs31 * gc13)) * inv_nm1
        ocov = (mo - (os31 * oc13)) * inv_nm1
        dcv = gcov - ocov
        covf_ref[0, 0, 0] = jnp.sqrt(jnp.sum(dcv * dcv))
        gcen = gt - gc31
        ocen = ot - oc31
        gn2 = jnp.sum(gcen * gcen, axis=0, keepdims=True)
        on2 = jnp.sum(ocen * ocen, axis=0, keepdims=True)
        sgd_ref[0, 0, 0] = jnp.sum(jnp.sqrt(gn2))
        sgd2_ref[0, 0, 0] = jnp.sum(gn2)
        sod_ref[0, 0, 0] = jnp.sum(jnp.sqrt(on2))
        sod2_ref[0, 0, 0] = jnp.sum(on2)

    row0 = rb * RBLK
    o_blk = o_ref[0, pl.ds(row0, RBLK), :]
    g_blk = g_ref[0, pl.ds(row0, RBLK), :]
    h_blk = g_blk - o_blk
    ht = gt - ot

    def _cdist2(blk, full_t):
        cross = jax.lax.dot_general(blk, full_t, (((1,), (0,)), ((), ())),
                                    preferred_element_type=jnp.float32)
        sq_blk = jnp.sum(blk * blk, axis=1, keepdims=True)
        sq_full = jnp.sum(full_t * full_t, axis=0, keepdims=True)
        return jnp.maximum(sq_blk + sq_full - 2.0 * cross, 0.0)

    d2o = _cdist2(o_blk, ot)
    pmat = _cdist2(h_blk, ht)
    d2g = _cdist2(g_blk, gt)

    def _kth_threshold(mat, k):
        sel = mat
        for _ in range(k - 1):
            m = jnp.min(sel, axis=1, keepdims=True)
            sel = jnp.where(sel <= m, BIG, sel)
        return jnp.min(sel, axis=1, keepdims=True)

    t17 = _kth_threshold(d2o, 17)
    local_ref[0, 0] += jnp.sum(jnp.where(d2o <= t17, pmat, 0.0))

    sel = d2g
    srow = None
    for i in range(5):
        m = jnp.min(sel, axis=1, keepdims=True)
        if i > 0:
            srow = jnp.sqrt(m) if srow is None else srow + jnp.sqrt(m)
        if i < 4:
            sel = jnp.where(sel <= m, BIG, sel)
    md = srow * 0.25
    smd_ref[0, 0, 0] += jnp.sum(md)
    smd2_ref[0, 0, 0] += jnp.sum(md * md)


def kernel(pred_noise, target_noise, generated_points, original_points,
           content_original, content_from_noisy, style_source, style_target):
    gt = jnp.transpose(generated_points, (0, 2, 1))
    ot = jnp.transpose(original_points, (0, 2, 1))
    ss = style_source[:, None, :]
    st = style_target[:, None, :]

    smem11 = pl.BlockSpec((1, 1), lambda b, rb: (0, 0), memory_space=pltpu.SMEM)
    smemb1 = pl.BlockSpec((1, 1, 1), lambda b, rb: (b, 0, 0),
                          memory_space=pltpu.SMEM)
    f32 = jnp.float32

    out_shape = ([jax.ShapeDtypeStruct((1, 1), f32)] * 7
                 + [jax.ShapeDtypeStruct((B, 1, 1), f32)] * 9)
    out_specs = [smem11] * 7 + [smemb1] * 9

    in_specs = [
        pl.BlockSpec((1, RBLK, 3), lambda b, rb: (b, rb, 0)),
        pl.BlockSpec((1, RBLK, 3), lambda b, rb: (b, rb, 0)),
        pl.BlockSpec((1, NPTS, 3), lambda b, rb: (b, 0, 0)),
        pl.BlockSpec((1, NPTS, 3), lambda b, rb: (b, 0, 0)),
        pl.BlockSpec((1, 3, NPTS), lambda b, rb: (b, 0, 0)),
        pl.BlockSpec((1, 3, NPTS), lambda b, rb: (b, 0, 0)),
        pl.BlockSpec((1, CCH, CDIM), lambda b, rb: (b, rb, 0)),
        pl.BlockSpec((1, CCH, CDIM), lambda b, rb: (b, rb, 0)),
        pl.BlockSpec((1, 1, SDIM), lambda b, rb: (b, 0, 0)),
        pl.BlockSpec((1, 1, SDIM), lambda b, rb: (b, 0, 0)),
    ]

    outs = pl.pallas_call(
        _loss_body,
        grid=(B, NBLK),
        in_specs=in_specs,
        out_specs=out_specs,
        out_shape=out_shape,
    )(pred_noise, target_noise, generated_points, original_points,
      gt, ot, content_original, content_from_noisy, ss, st)

    (diff2, cmse, varo, varn, abso, style, local,
     smd, smd2, cen, rng, covf, sgd, sgd2, sod, sod2) = outs

    diff_loss = diff2[0, 0] / (B * NPTS * 3)

    c_mse = cmse[0, 0] / (B * CROWS * CDIM)
    orig_var = varo[0, 0] / (B * CROWS)
    noisy_var = varn[0, 0] / (B * CROWS)
    var_loss = jax.nn.relu(0.1 - orig_var) + jax.nn.relu(0.1 - noisy_var)
    act_loss = jax.nn.relu(1.0 - abso[0, 0] / (B * CROWS * CDIM)) * 0.1
    content_loss = c_mse + var_loss + act_loss

    center_loss = jnp.sum(cen) / (B * 3)
    range_loss = jnp.sum(rng) / (B * 3)
    cov_loss = jnp.mean(covf)
    mgd = sgd[:, 0, 0] / NPTS
    mod = sod[:, 0, 0] / NPTS
    stdg = jnp.sqrt((sgd2[:, 0, 0] - sgd[:, 0, 0] ** 2 / NPTS) / (NPTS - 1))
    stdo = jnp.sqrt((sod2[:, 0, 0] - sod[:, 0, 0] ** 2 / NPTS) / (NPTS - 1))
    dist_loss = jnp.mean((mgd - mod) ** 2) + jnp.mean((stdg - stdo) ** 2)
    shape_loss = center_loss + range_loss * 0.5 + cov_loss * 0.1 + dist_loss * 0.5

    local_loss = local[0, 0] / (B * NPTS * 16 * 3)

    mmd = smd[:, 0, 0] / NPTS
    stds = jnp.sqrt((smd2[:, 0, 0] - smd[:, 0, 0] ** 2 / NPTS) / (NPTS - 1))
    smooth_loss = jnp.mean(stds)
    del mmd

    style_loss = style[0, 0] / B

    total = (diff_loss + content_loss + 2.0 * shape_loss
             + local_loss + 0.5 * smooth_loss + 0.1 * style_loss)
    return total


# speedup vs baseline: 33.0345x; 1.0718x over previous
import jax
import jax.numpy as jnp
from jax.experimental import pallas as pl
from jax.experimental.pallas import tpu as pltpu

B = 8
NPTS = 2048
CROWS = 256
CDIM = 2048
SDIM = 256
RBLK = 1024
NBLK = NPTS // RBLK
CCH = CROWS // NBLK
BIG = 3.0e38


def _loss_body(pred_ref, targ_ref, g_ref, o_ref, gt_ref, ot_ref, co_ref, cn_ref,
               ss_ref, st_ref,
               diff2_ref, cmse_ref, varo_ref, varn_ref, abso_ref, style_ref,
               local_ref, smd_ref, smd2_ref, cen_ref, rng_ref, covf_ref,
               sgd_ref, sgd2_ref, sod_ref, sod2_ref):
    b = pl.program_id(0)
    rb = pl.program_id(1)

    @pl.when(jnp.logical_and(b == 0, rb == 0))
    def _init_global():
        for r in (diff2_ref, cmse_ref, varo_ref, varn_ref, abso_ref,
                  style_ref, local_ref):
            r[0, 0] = 0.0

    @pl.when(rb == 0)
    def _init_batch():
        for r in (smd_ref, smd2_ref, cen_ref, rng_ref, covf_ref,
                  sgd_ref, sgd2_ref, sod_ref, sod2_ref):
            r[0, 0, 0] = 0.0

    dd = pred_ref[0] - targ_ref[0]
    diff2_ref[0, 0] += jnp.sum(dd * dd)

    co = co_ref[0]
    cn = cn_ref[0]
    dc = cn - co
    cmse_ref[0, 0] += jnp.sum(dc * dc)
    so = jnp.sum(co, axis=1)
    so2 = jnp.sum(co * co, axis=1)
    varo_ref[0, 0] += jnp.sum((so2 - so * so * (1.0 / CDIM)) * (1.0 / (CDIM - 1)))
    sn = jnp.sum(cn, axis=1)
    sn2 = jnp.sum(cn * cn, axis=1)
    varn_ref[0, 0] += jnp.sum((sn2 - sn * sn * (1.0 / CDIM)) * (1.0 / (CDIM - 1)))
    abso_ref[0, 0] += jnp.sum(jnp.abs(co))

    @pl.when(rb == 0)
    def _style():
        s = ss_ref[0]
        t = st_ref[0]
        num = jnp.sum(s * t)
        ns = jnp.maximum(jnp.sqrt(jnp.sum(s * s)), 1e-8)
        nt = jnp.maximum(jnp.sqrt(jnp.sum(t * t)), 1e-8)
        style_ref[0, 0] += jnp.abs(num / (ns * nt) - 0.5)

    gt = gt_ref[0]
    ot = ot_ref[0]
    g_full = g_ref[0]
    o_full = o_ref[0]

    @pl.when(rb == 0)
    def _moments():
        gs31 = jnp.sum(gt, axis=1, keepdims=True)
        os31 = jnp.sum(ot, axis=1, keepdims=True)
        gs13 = jnp.sum(g_full, axis=0, keepdims=True)
        os13 = jnp.sum(o_full, axis=0, keepdims=True)
        inv_n = 1.0 / NPTS
        gc31, oc31 = gs31 * inv_n, os31 * inv_n
        gc13, oc13 = gs13 * inv_n, os13 * inv_n
        dcn = gc31 - oc31
        cen_ref[0, 0, 0] = jnp.sum(dcn * dcn)
        rg = (jnp.max(gt, axis=1, keepdims=True) - jnp.min(gt, axis=1, keepdims=True)) \
           - (jnp.max(ot, axis=1, keepdims=True) - jnp.min(ot, axis=1, keepdims=True))
        rng_ref[0, 0, 0] = jnp.sum(rg * rg)
        mg = jax.lax.dot_general(gt, g_full, (((1,), (0,)), ((), ())),
                                 preferred_element_type=jnp.float32)
        mo = jax.lax.dot_general(ot, o_full, (((1,), (0,)), ((), ())),
                                 preferred_element_type=jnp.float32)
        inv_nm1 = 1.0 / (NPTS - 1)
        gcov = (mg - (gs31 * gc13)) * inv_nm1
        ocov = (mo - (os31 * oc13)) * inv_nm1
        dcv = gcov - ocov
        covf_ref[0, 0, 0] = jnp.sqrt(jnp.sum(dcv * dcv))
        gcen = gt - gc31
        ocen = ot - oc31
        gn2 = jnp.sum(gcen * gcen, axis=0, keepdims=True)
        on2 = jnp.sum(ocen * ocen, axis=0, keepdims=True)
        sgd_ref[0, 0, 0] = jnp.sum(jnp.sqrt(gn2))
        sgd2_ref[0, 0, 0] = jnp.sum(gn2)
        sod_ref[0, 0, 0] = jnp.sum(jnp.sqrt(on2))
        sod2_ref[0, 0, 0] = jnp.sum(on2)

    row0 = rb * RBLK
    o_blk = o_ref[0, pl.ds(row0, RBLK), :]
    g_blk = g_ref[0, pl.ds(row0, RBLK), :]
    h_blk = g_blk - o_blk
    ht = gt - ot

    def _cdist2(blk, full_t):
        cross = jax.lax.dot_general(blk, full_t, (((1,), (0,)), ((), ())),
                                    preferred_element_type=jnp.float32)
        sq_blk = jnp.sum(blk * blk, axis=1, keepdims=True)
        sq_full = jnp.sum(full_t * full_t, axis=0, keepdims=True)
        return jnp.maximum(sq_blk + sq_full - 2.0 * cross, 0.0)

    d2o = _cdist2(o_blk, ot)
    pmat = _cdist2(h_blk, ht)
    d2g = _cdist2(g_blk, gt)

    def _kth_threshold(mat, k):
        sel = mat
        for _ in range(k - 1):
            m = jnp.min(sel, axis=1, keepdims=True)
            sel = jnp.where(sel <= m, BIG, sel)
        return jnp.min(sel, axis=1, keepdims=True)

    t17 = _kth_threshold(d2o, 17)
    local_ref[0, 0] += jnp.sum(jnp.where(d2o <= t17, pmat, 0.0))

    sel = d2g
    srow = None
    for i in range(5):
        m = jnp.min(sel, axis=1, keepdims=True)
        if i > 0:
            srow = jnp.sqrt(m) if srow is None else srow + jnp.sqrt(m)
        if i < 4:
            sel = jnp.where(sel <= m, BIG, sel)
    md = srow * 0.25
    smd_ref[0, 0, 0] += jnp.sum(md)
    smd2_ref[0, 0, 0] += jnp.sum(md * md)


def kernel(pred_noise, target_noise, generated_points, original_points,
           content_original, content_from_noisy, style_source, style_target):
    gt = jnp.transpose(generated_points, (0, 2, 1))
    ot = jnp.transpose(original_points, (0, 2, 1))
    ss = style_source[:, None, :]
    st = style_target[:, None, :]

    smem11 = pl.BlockSpec((1, 1), lambda b, rb: (0, 0), memory_space=pltpu.SMEM)
    smemb1 = pl.BlockSpec((1, 1, 1), lambda b, rb: (b, 0, 0),
                          memory_space=pltpu.SMEM)
    f32 = jnp.float32

    out_shape = ([jax.ShapeDtypeStruct((1, 1), f32)] * 7
                 + [jax.ShapeDtypeStruct((B, 1, 1), f32)] * 9)
    out_specs = [smem11] * 7 + [smemb1] * 9

    in_specs = [
        pl.BlockSpec((1, RBLK, 3), lambda b, rb: (b, rb, 0)),
        pl.BlockSpec((1, RBLK, 3), lambda b, rb: (b, rb, 0)),
        pl.BlockSpec((1, NPTS, 3), lambda b, rb: (b, 0, 0)),
        pl.BlockSpec((1, NPTS, 3), lambda b, rb: (b, 0, 0)),
        pl.BlockSpec((1, 3, NPTS), lambda b, rb: (b, 0, 0)),
        pl.BlockSpec((1, 3, NPTS), lambda b, rb: (b, 0, 0)),
        pl.BlockSpec((1, CCH, CDIM), lambda b, rb: (b, rb, 0)),
        pl.BlockSpec((1, CCH, CDIM), lambda b, rb: (b, rb, 0)),
        pl.BlockSpec((1, 1, SDIM), lambda b, rb: (b, 0, 0)),
        pl.BlockSpec((1, 1, SDIM), lambda b, rb: (b, 0, 0)),
    ]

    outs = pl.pallas_call(
        _loss_body,
        grid=(B, NBLK),
        in_specs=in_specs,
        out_specs=out_specs,
        out_shape=out_shape,
    )(pred_noise, target_noise, generated_points, original_points,
      gt, ot, content_original, content_from_noisy, ss, st)

    (diff2, cmse, varo, varn, abso, style, local,
     smd, smd2, cen, rng, covf, sgd, sgd2, sod, sod2) = outs

    diff_loss = diff2[0, 0] / (B * NPTS * 3)

    c_mse = cmse[0, 0] / (B * CROWS * CDIM)
    orig_var = varo[0, 0] / (B * CROWS)
    noisy_var = varn[0, 0] / (B * CROWS)
    var_loss = jax.nn.relu(0.1 - orig_var) + jax.nn.relu(0.1 - noisy_var)
    act_loss = jax.nn.relu(1.0 - abso[0, 0] / (B * CROWS * CDIM)) * 0.1
    content_loss = c_mse + var_loss + act_loss

    center_loss = jnp.sum(cen) / (B * 3)
    range_loss = jnp.sum(rng) / (B * 3)
    cov_loss = jnp.mean(covf)
    mgd = sgd[:, 0, 0] / NPTS
    mod = sod[:, 0, 0] / NPTS
    stdg = jnp.sqrt((sgd2[:, 0, 0] - sgd[:, 0, 0] ** 2 / NPTS) / (NPTS - 1))
    stdo = jnp.sqrt((sod2[:, 0, 0] - sod[:, 0, 0] ** 2 / NPTS) / (NPTS - 1))
    dist_loss = jnp.mean((mgd - mod) ** 2) + jnp.mean((stdg - stdo) ** 2)
    shape_loss = center_loss + range_loss * 0.5 + cov_loss * 0.1 + dist_loss * 0.5

    local_loss = local[0, 0] / (B * NPTS * 16 * 3)

    mmd = smd[:, 0, 0] / NPTS
    stds = jnp.sqrt((smd2[:, 0, 0] - smd[:, 0, 0] ** 2 / NPTS) / (NPTS - 1))
    smooth_loss = jnp.mean(stds)
    del mmd

    style_loss = style[0, 0] / B

    total = (diff_loss + content_loss + 2.0 * shape_loss
             + local_loss + 0.5 * smooth_loss + 0.1 * style_loss)
    return total


# speedup vs baseline: 33.6995x; 1.0201x over previous
import jax
import jax.numpy as jnp
from jax.experimental import pallas as pl
from jax.experimental.pallas import tpu as pltpu

B = 8
NPTS = 2048
CROWS = 256
CDIM = 2048
SDIM = 256
RBLK = 1024
NBLK = NPTS // RBLK
CCH = CROWS // NBLK
BIG = 3.0e38


def _loss_body(pred_ref, targ_ref, g_ref, o_ref, gt_ref, ot_ref, co_ref, cn_ref,
               ss_ref, st_ref,
               diff2_ref, cmse_ref, varo_ref, varn_ref, abso_ref, style_ref,
               local_ref, smd_ref, smd2_ref, cen_ref, rng_ref, covf_ref,
               sgd_ref, sgd2_ref, sod_ref, sod2_ref):
    b = pl.program_id(0)
    rb = pl.program_id(1)

    @pl.when(jnp.logical_and(b == 0, rb == 0))
    def _init_global():
        for r in (diff2_ref, cmse_ref, varo_ref, varn_ref, abso_ref,
                  style_ref, local_ref):
            r[0, 0] = 0.0

    @pl.when(rb == 0)
    def _init_batch():
        for r in (smd_ref, smd2_ref, cen_ref, rng_ref, covf_ref,
                  sgd_ref, sgd2_ref, sod_ref, sod2_ref):
            r[0, 0, 0] = 0.0

    dd = pred_ref[0] - targ_ref[0]
    diff2_ref[0, 0] += jnp.sum(dd * dd)

    co = co_ref[0]
    cn = cn_ref[0]
    dc = cn - co
    cmse_ref[0, 0] += jnp.sum(dc * dc)
    so = jnp.sum(co, axis=1)
    so2 = jnp.sum(co * co, axis=1)
    varo_ref[0, 0] += jnp.sum((so2 - so * so * (1.0 / CDIM)) * (1.0 / (CDIM - 1)))
    sn = jnp.sum(cn, axis=1)
    sn2 = jnp.sum(cn * cn, axis=1)
    varn_ref[0, 0] += jnp.sum((sn2 - sn * sn * (1.0 / CDIM)) * (1.0 / (CDIM - 1)))
    abso_ref[0, 0] += jnp.sum(jnp.abs(co))

    @pl.when(rb == 0)
    def _style():
        s = ss_ref[0]
        t = st_ref[0]
        num = jnp.sum(s * t)
        ns = jnp.maximum(jnp.sqrt(jnp.sum(s * s)), 1e-8)
        nt = jnp.maximum(jnp.sqrt(jnp.sum(t * t)), 1e-8)
        style_ref[0, 0] += jnp.abs(num / (ns * nt) - 0.5)

    gt = gt_ref[0]
    ot = ot_ref[0]
    g_full = g_ref[0]
    o_full = o_ref[0]

    @pl.when(rb == 0)
    def _moments():
        gs31 = jnp.sum(gt, axis=1, keepdims=True)
        os31 = jnp.sum(ot, axis=1, keepdims=True)
        gs13 = jnp.sum(g_full, axis=0, keepdims=True)
        os13 = jnp.sum(o_full, axis=0, keepdims=True)
        inv_n = 1.0 / NPTS
        gc31, oc31 = gs31 * inv_n, os31 * inv_n
        gc13, oc13 = gs13 * inv_n, os13 * inv_n
        dcn = gc31 - oc31
        cen_ref[0, 0, 0] = jnp.sum(dcn * dcn)
        rg = (jnp.max(gt, axis=1, keepdims=True) - jnp.min(gt, axis=1, keepdims=True)) \
           - (jnp.max(ot, axis=1, keepdims=True) - jnp.min(ot, axis=1, keepdims=True))
        rng_ref[0, 0, 0] = jnp.sum(rg * rg)
        mg = jax.lax.dot_general(gt, g_full, (((1,), (0,)), ((), ())),
                                 preferred_element_type=jnp.float32)
        mo = jax.lax.dot_general(ot, o_full, (((1,), (0,)), ((), ())),
                                 preferred_element_type=jnp.float32)
        inv_nm1 = 1.0 / (NPTS - 1)
        gcov = (mg - (gs31 * gc13)) * inv_nm1
        ocov = (mo - (os31 * oc13)) * inv_nm1
        dcv = gcov - ocov
        covf_ref[0, 0, 0] = jnp.sqrt(jnp.sum(dcv * dcv))
        gcen = gt - gc31
        ocen = ot - oc31
        gn2 = jnp.sum(gcen * gcen, axis=0, keepdims=True)
        on2 = jnp.sum(ocen * ocen, axis=0, keepdims=True)
        sgd_ref[0, 0, 0] = jnp.sum(jnp.sqrt(gn2))
        sgd2_ref[0, 0, 0] = jnp.sum(gn2)
        sod_ref[0, 0, 0] = jnp.sum(jnp.sqrt(on2))
        sod2_ref[0, 0, 0] = jnp.sum(on2)

    row0 = rb * RBLK
    o_blk = o_ref[0, pl.ds(row0, RBLK), :]
    g_blk = g_ref[0, pl.ds(row0, RBLK), :]
    h_blk = g_blk - o_blk
    ht = gt - ot

    def _cdist2(blk, full_t):
        cross = jax.lax.dot_general(blk, full_t, (((1,), (0,)), ((), ())),
                                    preferred_element_type=jnp.float32)
        sq_blk = jnp.sum(blk * blk, axis=1, keepdims=True)
        sq_full = jnp.sum(full_t * full_t, axis=0, keepdims=True)
        return jnp.maximum(sq_blk + sq_full - 2.0 * cross, 0.0)

    d2o = _cdist2(o_blk, ot)
    pmat = _cdist2(h_blk, ht)
    d2g = _cdist2(g_blk, gt)

    def _kth_threshold(mat, k):
        m = jnp.min(mat, axis=1, keepdims=True)
        for _ in range(k - 1):
            m = jnp.min(jnp.where(mat > m, mat, BIG), axis=1, keepdims=True)
        return m

    t17 = _kth_threshold(d2o, 17)
    local_ref[0, 0] += jnp.sum(jnp.where(d2o <= t17, pmat, 0.0))

    m = jnp.min(d2g, axis=1, keepdims=True)
    srow = None
    for _ in range(4):
        m = jnp.min(jnp.where(d2g > m, d2g, BIG), axis=1, keepdims=True)
        srow = jnp.sqrt(m) if srow is None else srow + jnp.sqrt(m)
    md = srow * 0.25
    smd_ref[0, 0, 0] += jnp.sum(md)
    smd2_ref[0, 0, 0] += jnp.sum(md * md)


def kernel(pred_noise, target_noise, generated_points, original_points,
           content_original, content_from_noisy, style_source, style_target):
    gt = jnp.transpose(generated_points, (0, 2, 1))
    ot = jnp.transpose(original_points, (0, 2, 1))
    ss = style_source[:, None, :]
    st = style_target[:, None, :]

    smem11 = pl.BlockSpec((1, 1), lambda b, rb: (0, 0), memory_space=pltpu.SMEM)
    smemb1 = pl.BlockSpec((1, 1, 1), lambda b, rb: (b, 0, 0),
                          memory_space=pltpu.SMEM)
    f32 = jnp.float32

    out_shape = ([jax.ShapeDtypeStruct((1, 1), f32)] * 7
                 + [jax.ShapeDtypeStruct((B, 1, 1), f32)] * 9)
    out_specs = [smem11] * 7 + [smemb1] * 9

    in_specs = [
        pl.BlockSpec((1, RBLK, 3), lambda b, rb: (b, rb, 0)),
        pl.BlockSpec((1, RBLK, 3), lambda b, rb: (b, rb, 0)),
        pl.BlockSpec((1, NPTS, 3), lambda b, rb: (b, 0, 0)),
        pl.BlockSpec((1, NPTS, 3), lambda b, rb: (b, 0, 0)),
        pl.BlockSpec((1, 3, NPTS), lambda b, rb: (b, 0, 0)),
        pl.BlockSpec((1, 3, NPTS), lambda b, rb: (b, 0, 0)),
        pl.BlockSpec((1, CCH, CDIM), lambda b, rb: (b, rb, 0)),
        pl.BlockSpec((1, CCH, CDIM), lambda b, rb: (b, rb, 0)),
        pl.BlockSpec((1, 1, SDIM), lambda b, rb: (b, 0, 0)),
        pl.BlockSpec((1, 1, SDIM), lambda b, rb: (b, 0, 0)),
    ]

    outs = pl.pallas_call(
        _loss_body,
        grid=(B, NBLK),
        in_specs=in_specs,
        out_specs=out_specs,
        out_shape=out_shape,
    )(pred_noise, target_noise, generated_points, original_points,
      gt, ot, content_original, content_from_noisy, ss, st)

    (diff2, cmse, varo, varn, abso, style, local,
     smd, smd2, cen, rng, covf, sgd, sgd2, sod, sod2) = outs

    diff_loss = diff2[0, 0] / (B * NPTS * 3)

    c_mse = cmse[0, 0] / (B * CROWS * CDIM)
    orig_var = varo[0, 0] / (B * CROWS)
    noisy_var = varn[0, 0] / (B * CROWS)
    var_loss = jax.nn.relu(0.1 - orig_var) + jax.nn.relu(0.1 - noisy_var)
    act_loss = jax.nn.relu(1.0 - abso[0, 0] / (B * CROWS * CDIM)) * 0.1
    content_loss = c_mse + var_loss + act_loss

    center_loss = jnp.sum(cen) / (B * 3)
    range_loss = jnp.sum(rng) / (B * 3)
    cov_loss = jnp.mean(covf)
    mgd = sgd[:, 0, 0] / NPTS
    mod = sod[:, 0, 0] / NPTS
    stdg = jnp.sqrt((sgd2[:, 0, 0] - sgd[:, 0, 0] ** 2 / NPTS) / (NPTS - 1))
    stdo = jnp.sqrt((sod2[:, 0, 0] - sod[:, 0, 0] ** 2 / NPTS) / (NPTS - 1))
    dist_loss = jnp.mean((mgd - mod) ** 2) + jnp.mean((stdg - stdo) ** 2)
    shape_loss = center_loss + range_loss * 0.5 + cov_loss * 0.1 + dist_loss * 0.5

    local_loss = local[0, 0] / (B * NPTS * 16 * 3)

    mmd = smd[:, 0, 0] / NPTS
    stds = jnp.sqrt((smd2[:, 0, 0] - smd[:, 0, 0] ** 2 / NPTS) / (NPTS - 1))
    smooth_loss = jnp.mean(stds)
    del mmd

    style_loss = style[0, 0] / B

    total = (diff_loss + content_loss + 2.0 * shape_loss
             + local_loss + 0.5 * smooth_loss + 0.1 * style_loss)
    return total
